# Initial kernel scaffold; baseline (speedup 1.0000x reference)
import jax
import jax.numpy as jnp
from jax.experimental import pallas as pl


def kernel(x, edge_attr, edge_index, edge_type, A, B, C):
    raise NotImplementedError("write your pallas kernel here")


# speedup vs baseline: 1.3027x; 1.3027x over previous
import functools

import jax
import jax.numpy as jnp
from jax import lax
from jax.experimental import pallas as pl
from jax.experimental.pallas import tpu as pltpu
from jax.experimental.pallas import tpu_sc as plsc

N_NODES = 10000
N_EDGES = 160000
D_IN = 256
D_E = 16
D_OUT = 256

_NC, _NS = 2, 16
_NW = _NC * _NS
_EPW = N_EDGES // _NW
_CB = 40
_NCHUNK = _EPW // _CB


def _proj_body(x_ref, a_ref, b_ref, hi_ref, hj_ref):
    xb = x_ref[...]
    hi_ref[...] = jnp.dot(xb, a_ref[...], preferred_element_type=jnp.float32)
    hj_ref[...] = jnp.dot(xb, b_ref[...], preferred_element_type=jnp.float32)


def _edge_proj_body(ea_ref, c_ref, ec_ref):
    ec_ref[...] = jnp.dot(ea_ref[...], c_ref[...],
                          preferred_element_type=jnp.float32)


def _node_proj(x, A, B):
    blk = 1000
    return pl.pallas_call(
        _proj_body,
        grid=(N_NODES // blk,),
        in_specs=[
            pl.BlockSpec((blk, D_IN), lambda i: (i, 0)),
            pl.BlockSpec((D_IN, D_OUT), lambda i: (0, 0)),
            pl.BlockSpec((D_IN, D_OUT), lambda i: (0, 0)),
        ],
        out_specs=[
            pl.BlockSpec((blk, D_OUT), lambda i: (i, 0)),
            pl.BlockSpec((blk, D_OUT), lambda i: (i, 0)),
        ],
        out_shape=[jax.ShapeDtypeStruct((N_NODES, D_OUT), jnp.float32)] * 2,
    )(x, A, B)


def _edge_proj(edge_attr, C):
    blk = 2000
    return pl.pallas_call(
        _edge_proj_body,
        grid=(N_EDGES // blk,),
        in_specs=[
            pl.BlockSpec((blk, D_E), lambda i: (i, 0)),
            pl.BlockSpec((D_E, D_OUT), lambda i: (0, 0)),
        ],
        out_specs=pl.BlockSpec((blk, D_OUT), lambda i: (i, 0)),
        out_shape=jax.ShapeDtypeStruct((N_EDGES, D_OUT), jnp.float32),
    )(edge_attr, C)


def _sc_body(hi_hbm, hj_hbm, row_hbm, col_hbm, ec_hbm, out_hbm,
             idx_i, idx_j, bi, bj, be, sem_i, sem_j):
    wid = lax.axis_index("s") * _NC + lax.axis_index("c")
    wbase = wid * _EPW

    def chunk(k, carry):
        base = wbase + k * _CB
        pltpu.sync_copy(row_hbm.at[pl.ds(base, _CB)], idx_i)
        pltpu.sync_copy(col_hbm.at[pl.ds(base, _CB)], idx_j)
        cp_i = pltpu.async_copy(hi_hbm.at[idx_i], bi, sem_i)
        cp_j = pltpu.async_copy(hj_hbm.at[idx_j], bj, sem_j)
        pltpu.sync_copy(ec_hbm.at[pl.ds(base, _CB)], be)
        cp_i.wait()
        cp_j.wait()

        def rowfn(e, c2):
            for c in range(D_OUT // 16):
                sl = pl.ds(c * 16, 16)
                v = bi[e, sl] + bj[e, sl] + be[e, sl]
                bi[e, sl] = jnp.maximum(v, 0.0)
            return c2

        lax.fori_loop(0, _CB, rowfn, 0, unroll=False)
        pltpu.sync_copy(bi, out_hbm.at[pl.ds(base, _CB)])
        return carry

    lax.fori_loop(0, _NCHUNK, chunk, 0, unroll=False)


def _sc_gather(hi, hj, row, col, ec):
    mesh = plsc.VectorSubcoreMesh(core_axis_name="c", subcore_axis_name="s",
                                  num_cores=_NC, num_subcores=_NS)
    f = pl.kernel(
        _sc_body,
        out_type=jax.ShapeDtypeStruct((N_EDGES, D_OUT), jnp.float32),
        mesh=mesh,
        scratch_types=[
            pltpu.VMEM((_CB,), jnp.int32),
            pltpu.VMEM((_CB,), jnp.int32),
            pltpu.VMEM((_CB, D_OUT), jnp.float32),
            pltpu.VMEM((_CB, D_OUT), jnp.float32),
            pltpu.VMEM((_CB, D_OUT), jnp.float32),
            pltpu.SemaphoreType.DMA,
            pltpu.SemaphoreType.DMA,
        ],
    )
    return f(hi, hj, row, col, ec)


def kernel(x, edge_attr, edge_index, edge_type, A, B, C):
    del edge_type
    row = edge_index[0]
    col = edge_index[1]
    hi, hj = _node_proj(x, A, B)
    ec = _edge_proj(edge_attr, C)
    return _sc_gather(hi, hj, row, col, ec)


# speedup vs baseline: 2.8820x; 2.2122x over previous
import functools

import numpy as np

import jax
import jax.numpy as jnp
from jax import lax
from jax.experimental import pallas as pl
from jax.experimental.pallas import tpu as pltpu
from jax.experimental.pallas import tpu_sc as plsc

N_NODES = 10000
N_EDGES = 160000
D_IN = 256
D_E = 16
D_OUT = 256
D_H = D_OUT // 2

_NC, _NS = 2, 16
_NW = _NC * _NS
_EPW = N_EDGES // _NW
_CB = 40
_NCHUNK = _EPW // _CB
_NSLOT = 5
_MASK = np.uint32(0xFFFF0000)


def _pack_bf16_pair(lo_f32, hi_f32):
    lo_bits = lax.bitcast_convert_type(
        lo_f32.astype(jnp.bfloat16).astype(jnp.float32), jnp.uint32)
    hi_bits = lax.bitcast_convert_type(
        hi_f32.astype(jnp.bfloat16).astype(jnp.float32), jnp.uint32)
    word = (lo_bits >> 16) | (hi_bits & _MASK)
    return lax.bitcast_convert_type(word, jnp.int32)


def _unpack_bf16_pair(word_i32):
    w = lax.bitcast_convert_type(word_i32, jnp.uint32)
    lo = lax.bitcast_convert_type(w << 16, jnp.float32)
    hi = lax.bitcast_convert_type(w & _MASK, jnp.float32)
    return lo, hi


def _proj_body(x_ref, a_ref, b_ref, hi_ref, hj_ref):
    xb = x_ref[...]
    hi = jnp.dot(xb, a_ref[...], preferred_element_type=jnp.float32)
    hj = jnp.dot(xb, b_ref[...], preferred_element_type=jnp.float32)
    hi_ref[...] = _pack_bf16_pair(hi[:, :D_H], hi[:, D_H:])
    hj_ref[...] = _pack_bf16_pair(hj[:, :D_H], hj[:, D_H:])


def _node_proj(x, A, B):
    blk = 1000
    return pl.pallas_call(
        _proj_body,
        grid=(N_NODES // blk,),
        in_specs=[
            pl.BlockSpec((blk, D_IN), lambda i: (i, 0)),
            pl.BlockSpec((D_IN, D_OUT), lambda i: (0, 0)),
            pl.BlockSpec((D_IN, D_OUT), lambda i: (0, 0)),
        ],
        out_specs=[
            pl.BlockSpec((blk, D_H), lambda i: (i, 0)),
            pl.BlockSpec((blk, D_H), lambda i: (i, 0)),
        ],
        out_shape=[jax.ShapeDtypeStruct((N_NODES, D_H), jnp.int32)] * 2,
    )(x, A, B)


def _fuse_body(gi_ref, gj_ref, ea_ref, c_ref, out_ref):
    ec = jnp.dot(ea_ref[...], c_ref[...], preferred_element_type=jnp.float32)
    gil, gih = _unpack_bf16_pair(gi_ref[...])
    gjl, gjh = _unpack_bf16_pair(gj_ref[...])
    out_ref[:, :D_H] = jnp.maximum(gil + gjl + ec[:, :D_H], 0.0)
    out_ref[:, D_H:] = jnp.maximum(gih + gjh + ec[:, D_H:], 0.0)


def _edge_fuse(gi, gj, edge_attr, C):
    blk = 2000
    return pl.pallas_call(
        _fuse_body,
        grid=(N_EDGES // blk,),
        in_specs=[
            pl.BlockSpec((blk, D_H), lambda i: (i, 0)),
            pl.BlockSpec((blk, D_H), lambda i: (i, 0)),
            pl.BlockSpec((blk, D_E), lambda i: (i, 0)),
            pl.BlockSpec((D_E, D_OUT), lambda i: (0, 0)),
        ],
        out_specs=pl.BlockSpec((blk, D_OUT), lambda i: (i, 0)),
        out_shape=jax.ShapeDtypeStruct((N_EDGES, D_OUT), jnp.float32),
    )(gi, gj, edge_attr, C)


def _sc_body(hi_hbm, hj_hbm, row_hbm, col_hbm, gi_hbm, gj_hbm,
             ii_all, ij_all, bufs_i, bufs_j, sems_gi, sems_gj,
             sems_oi, sems_oj):
    wid = lax.axis_index("s") * _NC + lax.axis_index("c")

    pltpu.sync_copy(row_hbm.at[wid], ii_all)
    pltpu.sync_copy(col_hbm.at[wid], ij_all)

    def issue(k, s):
        pltpu.async_copy(hi_hbm.at[ii_all.at[k]], bufs_i[s], sems_gi[s])
        pltpu.async_copy(hj_hbm.at[ij_all.at[k]], bufs_j[s], sems_gj[s])

    def finish(k, s):
        base = (wid * _EPW) + k * _CB
        pltpu.make_async_copy(hi_hbm.at[ii_all.at[k]], bufs_i[s],
                              sems_gi[s]).wait()
        pltpu.make_async_copy(hj_hbm.at[ij_all.at[k]], bufs_j[s],
                              sems_gj[s]).wait()
        pltpu.async_copy(bufs_i[s], gi_hbm.at[pl.ds(base, _CB)], sems_oi[s])
        pltpu.async_copy(bufs_j[s], gj_hbm.at[pl.ds(base, _CB)], sems_oj[s])

    def wait_out(s):
        pltpu.make_async_copy(bufs_i[s], gi_hbm.at[pl.ds(0, _CB)],
                              sems_oi[s]).wait()
        pltpu.make_async_copy(bufs_j[s], gj_hbm.at[pl.ds(0, _CB)],
                              sems_oj[s]).wait()

    issue(0, 0)
    issue(1, 1)
    issue(2, 2)

    def group(q, carry):
        k0 = _NSLOT * q
        for s in range(_NSLOT):
            k = k0 + s
            finish(k, s)
            nxt = k + 3
            ns = (s + 3) % _NSLOT

            @pl.when(nxt < _NCHUNK)
            def _():
                @pl.when(nxt >= _NSLOT)
                def _():
                    wait_out(ns)

                issue(nxt, ns)
        return carry

    lax.fori_loop(0, _NCHUNK // _NSLOT, group, 0, unroll=False)
    for s in range(_NSLOT):
        wait_out(s)


def _sc_gather(hi, hj, row3, col3):
    mesh = plsc.VectorSubcoreMesh(core_axis_name="c", subcore_axis_name="s",
                                  num_cores=_NC, num_subcores=_NS)
    f = pl.kernel(
        _sc_body,
        out_type=[jax.ShapeDtypeStruct((N_EDGES, D_H), jnp.int32)] * 2,
        mesh=mesh,
        scratch_types=[
            pltpu.VMEM((_NCHUNK, _CB), jnp.int32),
            pltpu.VMEM((_NCHUNK, _CB), jnp.int32),
            [pltpu.VMEM((_CB, D_H), jnp.int32) for _ in range(_NSLOT)],
            [pltpu.VMEM((_CB, D_H), jnp.int32) for _ in range(_NSLOT)],
            [pltpu.SemaphoreType.DMA for _ in range(_NSLOT)],
            [pltpu.SemaphoreType.DMA for _ in range(_NSLOT)],
            [pltpu.SemaphoreType.DMA for _ in range(_NSLOT)],
            [pltpu.SemaphoreType.DMA for _ in range(_NSLOT)],
        ],
    )
    return f(hi, hj, row3, col3)


def kernel(x, edge_attr, edge_index, edge_type, A, B, C):
    del edge_type
    row3 = edge_index[0].reshape(_NW, _NCHUNK, _CB)
    col3 = edge_index[1].reshape(_NW, _NCHUNK, _CB)
    hi, hj = _node_proj(x, A, B)
    gi, gj = _sc_gather(hi, hj, row3, col3)
    return _edge_fuse(gi, gj, edge_attr, C)


# speedup vs baseline: 3.5690x; 1.2384x over previous
import functools

import numpy as np

import jax
import jax.numpy as jnp
from jax import lax
from jax.experimental import pallas as pl
from jax.experimental.pallas import tpu as pltpu
from jax.experimental.pallas import tpu_sc as plsc

N_NODES = 10000
N_EDGES = 160000
D_IN = 256
D_E = 16
D_OUT = 256
D_H = D_OUT // 2

_NC, _NS = 2, 16
_NW = _NC * _NS
_EPW = N_EDGES // _NW
_EPT = N_EDGES // _NS
_CB = 40
_NCHUNK = _EPT // _CB
_NSLOT = 5
_MASK = np.uint32(0xFFFF0000)


def _pack_bf16_pair(lo_f32, hi_f32):
    lo_bits = lax.bitcast_convert_type(
        lo_f32.astype(jnp.bfloat16).astype(jnp.float32), jnp.uint32)
    hi_bits = lax.bitcast_convert_type(
        hi_f32.astype(jnp.bfloat16).astype(jnp.float32), jnp.uint32)
    word = (lo_bits >> 16) | (hi_bits & _MASK)
    return lax.bitcast_convert_type(word, jnp.int32)


def _unpack_bf16_pair(word_i32):
    w = lax.bitcast_convert_type(word_i32, jnp.uint32)
    lo = lax.bitcast_convert_type(w << 16, jnp.float32)
    hi = lax.bitcast_convert_type(w & _MASK, jnp.float32)
    return lo, hi


def _proj_body(x_ref, a_ref, b_ref, hi_ref, hj_ref):
    xb = x_ref[...]
    hi = jnp.dot(xb, a_ref[...], preferred_element_type=jnp.float32)
    hj = jnp.dot(xb, b_ref[...], preferred_element_type=jnp.float32)
    hi_ref[...] = _pack_bf16_pair(hi[:, :D_H], hi[:, D_H:])
    hj_ref[...] = _pack_bf16_pair(hj[:, :D_H], hj[:, D_H:])


def _node_proj(x, A, B):
    blk = 1000
    return pl.pallas_call(
        _proj_body,
        grid=(N_NODES // blk,),
        in_specs=[
            pl.BlockSpec((blk, D_IN), lambda i: (i, 0)),
            pl.BlockSpec((D_IN, D_OUT), lambda i: (0, 0)),
            pl.BlockSpec((D_IN, D_OUT), lambda i: (0, 0)),
        ],
        out_specs=[
            pl.BlockSpec((blk, D_H), lambda i: (i, 0)),
            pl.BlockSpec((blk, D_H), lambda i: (i, 0)),
        ],
        out_shape=[jax.ShapeDtypeStruct((N_NODES, D_H), jnp.int32)] * 2,
    )(x, A, B)


def _fuse_body(gi_ref, gj_ref, ea_ref, c_ref, out_ref):
    ec = jnp.dot(ea_ref[...], c_ref[...], preferred_element_type=jnp.float32)
    gil, gih = _unpack_bf16_pair(gi_ref[...])
    gjl, gjh = _unpack_bf16_pair(gj_ref[...])
    out_ref[:, :D_H] = jnp.maximum(gil + gjl + ec[:, :D_H], 0.0)
    out_ref[:, D_H:] = jnp.maximum(gih + gjh + ec[:, D_H:], 0.0)


def _edge_fuse(gi, gj, edge_attr, C):
    blk = 2000
    return pl.pallas_call(
        _fuse_body,
        grid=(N_EDGES // blk,),
        in_specs=[
            pl.BlockSpec((blk, D_H), lambda i: (i, 0)),
            pl.BlockSpec((blk, D_H), lambda i: (i, 0)),
            pl.BlockSpec((blk, D_E), lambda i: (i, 0)),
            pl.BlockSpec((D_E, D_OUT), lambda i: (0, 0)),
        ],
        out_specs=pl.BlockSpec((blk, D_OUT), lambda i: (i, 0)),
        out_shape=jax.ShapeDtypeStruct((N_EDGES, D_OUT), jnp.float32),
    )(gi, gj, edge_attr, C)


def _sc_body(hi_hbm, hj_hbm, row_hbm, col_hbm, gi_hbm, gj_hbm,
             shared, idx_all, bufs, sems_g, sems_o):
    cid = lax.axis_index("c")
    sid = lax.axis_index("s")

    def pipe(tab_hbm, idx_hbm, out_hbm):
        seg = 624
        pltpu.sync_copy(tab_hbm.at[pl.ds(sid * seg, seg)],
                        shared.at[pl.ds(sid * seg, seg)])

        @pl.when(sid == 0)
        def _():
            tail = N_NODES - seg * _NS
            pltpu.sync_copy(tab_hbm.at[pl.ds(seg * _NS, tail)],
                            shared.at[pl.ds(seg * _NS, tail)])

        pltpu.sync_copy(idx_hbm.at[pl.ds(sid * _EPT, _EPT)], idx_all)
        plsc.subcore_barrier()

        def issue(k, s):
            pltpu.async_copy(shared.at[idx_all.at[pl.ds(k * _CB, _CB)]],
                             bufs[s], sems_g[s])

        def finish(k, s):
            base = (sid * _EPT) + k * _CB
            pltpu.make_async_copy(shared.at[idx_all.at[pl.ds(k * _CB, _CB)]],
                                  bufs[s], sems_g[s]).wait()
            pltpu.async_copy(bufs[s], out_hbm.at[pl.ds(base, _CB)], sems_o[s])

        def wait_out(s):
            pltpu.make_async_copy(bufs[s], out_hbm.at[pl.ds(0, _CB)],
                                  sems_o[s]).wait()

        issue(0, 0)
        issue(1, 1)
        issue(2, 2)

        def group(q, carry):
            k0 = _NSLOT * q
            for s in range(_NSLOT):
                k = k0 + s
                finish(k, s)
                nxt = k + 3
                ns = (s + 3) % _NSLOT

                @pl.when(nxt < _NCHUNK)
                def _():
                    @pl.when(nxt >= _NSLOT)
                    def _():
                        wait_out(ns)

                    issue(nxt, ns)
            return carry

        lax.fori_loop(0, _NCHUNK // _NSLOT, group, 0, unroll=False)
        for s in range(_NSLOT):
            wait_out(s)

    @pl.when(cid == 0)
    def _():
        pipe(hi_hbm, row_hbm, gi_hbm)

    @pl.when(cid == 1)
    def _():
        pipe(hj_hbm, col_hbm, gj_hbm)


def _sc_gather(hi, hj, row3, col3):
    mesh = plsc.VectorSubcoreMesh(core_axis_name="c", subcore_axis_name="s",
                                  num_cores=_NC, num_subcores=_NS)
    f = pl.kernel(
        _sc_body,
        out_type=[jax.ShapeDtypeStruct((N_EDGES, D_H), jnp.int32)] * 2,
        mesh=mesh,
        scratch_types=[
            pltpu.VMEM_SHARED((N_NODES, D_H), jnp.int32),
            pltpu.VMEM((_EPT,), jnp.int32),
            [pltpu.VMEM((_CB, D_H), jnp.int32) for _ in range(_NSLOT)],
            [pltpu.SemaphoreType.DMA for _ in range(_NSLOT)],
            [pltpu.SemaphoreType.DMA for _ in range(_NSLOT)],
        ],
    )
    return f(hi, hj, row3, col3)


def kernel(x, edge_attr, edge_index, edge_type, A, B, C):
    del edge_type
    hi, hj = _node_proj(x, A, B)
    gi, gj = _sc_gather(hi, hj, edge_index[0], edge_index[1])
    return _edge_fuse(gi, gj, edge_attr, C)
